# baseline (speedup 1.0000x reference)
import functools

import jax
import jax.numpy as jnp
from jax import lax
from jax.experimental import pallas as pl
from jax.experimental.pallas import tpu as pltpu
from jax.experimental.pallas import tpu_sc as plsc

B, Lseq, V, D = 16384, 200, 100, 10
DP = 16
NB = B - 2
N = NB * Lseq
NW = 32
LANES = 16

CH = 2048
GP = CH // LANES
PW = 102400
NFULL = PW // CH - 1
TAIL_GROUPS = (N - 31 * PW - NFULL * CH) // LANES


def _normalize_table(table, gamma, beta):
    tpad = jnp.zeros((V, DP), jnp.float32).at[:, :D].set(table)
    gpad = jnp.zeros((1, DP), jnp.float32).at[0, :D].set(gamma)
    bpad = jnp.zeros((1, DP), jnp.float32).at[0, :D].set(beta)

    def body(t_ref, g_ref, b_ref, o_ref):
        x = t_ref[...]
        mean = jnp.sum(x, axis=-1, keepdims=True) * (1.0 / D)
        mask = lax.broadcasted_iota(jnp.int32, (V, DP), 1) < D
        cen = jnp.where(mask, x - mean, 0.0)
        var = jnp.sum(cen * cen, axis=-1, keepdims=True) * (1.0 / D)
        r = lax.rsqrt(var + 1e-5)
        o_ref[...] = cen * r * g_ref[...] + b_ref[...]

    return pl.pallas_call(
        body,
        out_shape=jax.ShapeDtypeStruct((V, DP), jnp.float32),
    )(tpad, gpad, bpad)


def _make_gather_kernel():
    mesh = plsc.VectorSubcoreMesh(core_axis_name="c", subcore_axis_name="s")

    @functools.partial(
        pl.kernel,
        out_type=jax.ShapeDtypeStruct((N * D,), jnp.float32),
        mesh=mesh,
        compiler_params=pltpu.CompilerParams(needs_layout_passes=False),
        scratch_types=[
            pltpu.VMEM((V * DP,), jnp.float32),
            pltpu.VMEM((CH,), jnp.int32),
            pltpu.VMEM((CH * D,), jnp.float32),
        ],
    )
    def gather_k(nt_hbm, ids_hbm, out_hbm, nt_v, ids_v, out_v):
        wid = lax.axis_index("s") * 2 + lax.axis_index("c")
        pltpu.sync_copy(nt_hbm, nt_v)
        base = Lseq + wid * PW
        iota10 = lax.iota(jnp.int32, LANES) * D

        def do_chunk(start_pos, ngroups):
            npos = ngroups * LANES
            pltpu.sync_copy(
                ids_hbm.at[pl.ds(start_pos, npos)],
                ids_v.at[pl.ds(0, npos)],
            )

            def group(g, _):
                idsg = ids_v[pl.ds(g * LANES, LANES)]
                row = idsg * DP
                dst0 = g * (LANES * D) + iota10
                for f in range(D):
                    vals = plsc.load_gather(nt_v, [row + f])
                    plsc.store_scatter(out_v, [dst0 + f], vals)
                return 0

            lax.fori_loop(0, ngroups, group, 0, unroll=2)
            pltpu.sync_copy(
                out_v.at[pl.ds(0, npos * D)],
                out_hbm.at[pl.ds((start_pos - Lseq) * D, npos * D)],
            )

        def chunk_body(c, _):
            do_chunk(base + c * CH, GP)
            return 0

        lax.fori_loop(0, NFULL, chunk_body, 0)

        tail_start = base + NFULL * CH

        @pl.when(wid < NW - 1)
        def _():
            do_chunk(tail_start, GP)

        @pl.when(wid == NW - 1)
        def _():
            do_chunk(tail_start, TAIL_GROUPS)

    return gather_k


_gather = _make_gather_kernel()


def kernel(input_ids, table, gamma, beta):
    nt = _normalize_table(table, gamma, beta).reshape(-1)
    ids_flat = input_ids.reshape(-1).astype(jnp.int32)
    out_flat = _gather(nt, ids_flat)
    scale = 1.0 + 0.0 * table[0, 0]
    return out_flat.reshape(NB, Lseq, D) * scale


# speedup vs baseline: 1.0005x; 1.0005x over previous
import functools

import jax
import jax.numpy as jnp
from jax import lax
from jax.experimental import pallas as pl
from jax.experimental.pallas import tpu as pltpu
from jax.experimental.pallas import tpu_sc as plsc

B, Lseq, V, D = 16384, 200, 100, 10
DP = 16
NB = B - 2
N = NB * Lseq
NW = 32
LANES = 16

CH = 2048
GP = CH // LANES
PW = 102400
NFULL = PW // CH - 1
TAIL_GROUPS = (N - 31 * PW - NFULL * CH) // LANES


def _normalize_table(table, gamma, beta):
    tpad = jnp.zeros((V, DP), jnp.float32).at[:, :D].set(table)
    gpad = jnp.zeros((1, DP), jnp.float32).at[0, :D].set(gamma)
    bpad = jnp.zeros((1, DP), jnp.float32).at[0, :D].set(beta)

    def body(t_ref, g_ref, b_ref, o_ref):
        x = t_ref[...]
        mean = jnp.sum(x, axis=-1, keepdims=True) * (1.0 / D)
        mask = lax.broadcasted_iota(jnp.int32, (V, DP), 1) < D
        cen = jnp.where(mask, x - mean, 0.0)
        var = jnp.sum(cen * cen, axis=-1, keepdims=True) * (1.0 / D)
        r = lax.rsqrt(var + 1e-5)
        o_ref[...] = cen * r * g_ref[...] + b_ref[...]

    return pl.pallas_call(
        body,
        out_shape=jax.ShapeDtypeStruct((V, DP), jnp.float32),
    )(tpad, gpad, bpad)


def _make_gather_kernel():
    mesh = plsc.VectorSubcoreMesh(core_axis_name="c", subcore_axis_name="s")

    @functools.partial(
        pl.kernel,
        out_type=jax.ShapeDtypeStruct((N * D,), jnp.float32),
        mesh=mesh,
        compiler_params=pltpu.CompilerParams(needs_layout_passes=False),
        scratch_types=[
            pltpu.VMEM((V * DP,), jnp.float32),
            pltpu.VMEM((CH,), jnp.int32),
            pltpu.VMEM((CH * D,), jnp.float32),
        ],
    )
    def gather_k(nt_hbm, ids_hbm, out_hbm, nt_v, ids_v, out_v):
        wid = lax.axis_index("s") * 2 + lax.axis_index("c")
        pltpu.sync_copy(nt_hbm, nt_v)
        base = Lseq + wid * PW
        iota10 = lax.iota(jnp.int32, LANES) * D

        def do_chunk(start_pos, ngroups):
            npos = ngroups * LANES
            pltpu.sync_copy(
                ids_hbm.at[pl.ds(start_pos, npos)],
                ids_v.at[pl.ds(0, npos)],
            )

            def group(g, _):
                idsg = ids_v[pl.ds(g * LANES, LANES)]
                row = idsg * DP
                dst0 = g * (LANES * D) + iota10
                for f in range(D):
                    vals = plsc.load_gather(nt_v, [row + f])
                    plsc.store_scatter(out_v, [dst0 + f], vals)
                return 0

            lax.fori_loop(0, ngroups, group, 0, unroll=2)
            pltpu.sync_copy(
                out_v.at[pl.ds(0, npos * D)],
                out_hbm.at[pl.ds((start_pos - Lseq) * D, npos * D)],
            )

        def chunk_body(c, _):
            do_chunk(base + c * CH, GP)
            return 0

        lax.fori_loop(0, NFULL, chunk_body, 0)

        tail_start = base + NFULL * CH

        @pl.when(wid < NW - 1)
        def _():
            do_chunk(tail_start, GP)

        @pl.when(wid == NW - 1)
        def _():
            do_chunk(tail_start, TAIL_GROUPS)

    return gather_k


_gather = _make_gather_kernel()


def kernel(input_ids, table, gamma, beta):
    nt = _normalize_table(table, gamma, beta).reshape(-1)
    ids_flat = input_ids.reshape(-1).astype(jnp.int32)
    out_flat = _gather(nt, ids_flat)
    scale = jnp.where(input_ids[0, 0] < jnp.int32(2**30), 1.0, 2.0)
    return out_flat.reshape(NB, Lseq, D) * scale


# speedup vs baseline: 6.6113x; 6.6080x over previous
import functools

import jax
import jax.numpy as jnp
from jax import lax
from jax.experimental import pallas as pl
from jax.experimental.pallas import tpu as pltpu
from jax.experimental.pallas import tpu_sc as plsc

B, Lseq, V, D = 16384, 200, 100, 10
DP = 16
NB = B - 2
NW = 32
LANES = 16

CB = 128
NBLK = B // CB
BPT = NBLK // NW
LT = Lseq // 8


def _normalize_table(table, gamma, beta):
    tpad = jnp.zeros((V, DP), jnp.float32).at[:, :D].set(table)
    gpad = jnp.zeros((1, DP), jnp.float32).at[0, :D].set(gamma)
    bpad = jnp.zeros((1, DP), jnp.float32).at[0, :D].set(beta)

    def body(t_ref, g_ref, b_ref, o_ref):
        x = t_ref[...]
        mean = jnp.sum(x, axis=-1, keepdims=True) * (1.0 / D)
        mask = lax.broadcasted_iota(jnp.int32, (V, DP), 1) < D
        cen = jnp.where(mask, x - mean, 0.0)
        var = jnp.sum(cen * cen, axis=-1, keepdims=True) * (1.0 / D)
        r = lax.rsqrt(var + 1e-5)
        o_ref[...] = cen * r * g_ref[...] + b_ref[...]

    return pl.pallas_call(
        body,
        out_shape=jax.ShapeDtypeStruct((V, DP), jnp.float32),
    )(tpad, gpad, bpad)


def _make_gather_kernel():
    mesh = plsc.VectorSubcoreMesh(core_axis_name="c", subcore_axis_name="s")

    @functools.partial(
        pl.kernel,
        out_type=jax.ShapeDtypeStruct((D, Lseq, B), jnp.float32),
        mesh=mesh,
        compiler_params=pltpu.CompilerParams(needs_layout_passes=False),
        scratch_types=[
            pltpu.VMEM((V * DP,), jnp.float32),
            pltpu.VMEM((CB * Lseq,), jnp.int32),
            pltpu.VMEM((D, 8, CB), jnp.float32),
        ],
    )
    def gather_k(nt_hbm, ids_hbm, out_hbm, nt_v, ids_v, out_v):
        wid = lax.axis_index("s") * 2 + lax.axis_index("c")
        pltpu.sync_copy(nt_hbm, nt_v)
        iota200 = lax.iota(jnp.int32, LANES) * Lseq

        for bb in range(BPT):
            blk = wid * BPT + bb
            b0 = blk * CB
            last = blk == NBLK - 1

            @pl.when(jnp.logical_not(last))
            def _():
                pltpu.sync_copy(
                    ids_hbm.at[pl.ds((b0 + 1) * Lseq, CB * Lseq)], ids_v)

            @pl.when(last)
            def _():
                pltpu.sync_copy(
                    ids_hbm.at[pl.ds((b0 + 1) * Lseq, (CB - 1) * Lseq)],
                    ids_v.at[pl.ds(0, (CB - 1) * Lseq)])

            def lt_body(lt, _):
                def l8_body(l8, _):
                    l = lt * 8 + l8
                    for sb in range(CB // LANES):
                        col = iota200 + (l + sb * (LANES * Lseq))
                        idsg = plsc.load_gather(ids_v, [col])
                        idsg = jnp.minimum(jnp.maximum(idsg, 0), V - 1)
                        rowb = idsg * DP
                        for f in range(D):
                            vals = plsc.load_gather(nt_v, [rowb + f])
                            out_v[f, l8, pl.ds(sb * LANES, LANES)] = vals
                    return 0

                lax.fori_loop(0, 8, l8_body, 0)
                pltpu.sync_copy(
                    out_v,
                    out_hbm.at[:, pl.ds(lt * 8, 8), pl.ds(b0, CB)])
                return 0

            lax.fori_loop(0, LT, lt_body, 0)

    return gather_k


_gather = _make_gather_kernel()


def kernel(input_ids, table, gamma, beta):
    nt = _normalize_table(table, gamma, beta).reshape(-1)
    ids_flat = input_ids.reshape(-1).astype(jnp.int32)
    out_t = _gather(nt, ids_flat)
    return jnp.transpose(out_t, (2, 1, 0))[:NB]


# speedup vs baseline: 43.1724x; 6.5301x over previous
import functools

import jax
import jax.numpy as jnp
from jax import lax
from jax.experimental import pallas as pl
from jax.experimental.pallas import tpu as pltpu
from jax.experimental.pallas import tpu_sc as plsc

B, Lseq, V, D = 16384, 200, 100, 10
DP = 16
NB = B - 2
NW = 32
LANES = 16

CB = 128
NBLK = B // CB
BPT = NBLK // NW
LT = Lseq // 8
SBS = CB // LANES


def _normalize_table(table, gamma, beta):
    tpad = jnp.zeros((V, DP), jnp.float32).at[:, :D].set(table)
    gpad = jnp.zeros((1, DP), jnp.float32).at[0, :D].set(gamma)
    bpad = jnp.zeros((1, DP), jnp.float32).at[0, :D].set(beta)

    def body(t_ref, g_ref, b_ref, o_ref):
        x = t_ref[...]
        mean = jnp.sum(x, axis=-1, keepdims=True) * (1.0 / D)
        mask = lax.broadcasted_iota(jnp.int32, (V, DP), 1) < D
        cen = jnp.where(mask, x - mean, 0.0)
        var = jnp.sum(cen * cen, axis=-1, keepdims=True) * (1.0 / D)
        r = lax.rsqrt(var + 1e-5)
        o_ref[...] = cen * r * g_ref[...] + b_ref[...]

    return pl.pallas_call(
        body,
        out_shape=jax.ShapeDtypeStruct((V, DP), jnp.float32),
    )(tpad, gpad, bpad)


def _make_gather_kernel():
    mesh = plsc.VectorSubcoreMesh(core_axis_name="c", subcore_axis_name="s")

    @functools.partial(
        pl.kernel,
        out_type=jax.ShapeDtypeStruct((D, Lseq, B), jnp.float32),
        mesh=mesh,
        compiler_params=pltpu.CompilerParams(needs_layout_passes=False),
        scratch_types=[
            pltpu.VMEM((V * DP,), jnp.float32),
            pltpu.VMEM((CB * Lseq,), jnp.int32),
            pltpu.VMEM((D, 8, CB), jnp.float32),
            pltpu.VMEM((D, 8, CB), jnp.float32),
            pltpu.SemaphoreType.DMA,
            pltpu.SemaphoreType.DMA,
        ],
    )
    def gather_k(nt_hbm, ids_hbm, out_hbm, nt_v, ids_v, out0, out1, so0, so1):
        wid = lax.axis_index("s") * 2 + lax.axis_index("c")
        pltpu.sync_copy(nt_hbm, nt_v)
        iota200 = lax.iota(jnp.int32, LANES) * Lseq

        def wait_out(out_v, sem):
            pltpu.make_async_copy(
                out_v, out_hbm.at[:, pl.ds(0, 8), pl.ds(0, CB)], sem).wait()

        for bb in range(BPT):
            blk = wid * BPT + bb
            b0 = blk * CB
            start_row = jnp.minimum(b0 + 1, B - CB)
            offs = (b0 + 1 - start_row) * Lseq
            pltpu.sync_copy(
                ids_hbm.at[pl.ds(start_row * Lseq, CB * Lseq)], ids_v)

            def compute(lt, out_v):
                @functools.partial(plsc.parallel_loop, 0, 8, unroll=2)
                def l8_body(l8):
                    l = lt * 8 + l8
                    for sb in range(SBS):
                        col = iota200 + (l + (sb * (LANES * Lseq)) + offs)
                        idsg = plsc.load_gather(ids_v, [col])
                        rowb = idsg * DP
                        for f in range(D):
                            vals = plsc.load_gather(nt_v, [rowb + f])
                            out_v[f, l8, pl.ds(sb * LANES, LANES)] = vals

            def start_out(lt, out_v, sem):
                pltpu.async_copy(
                    out_v,
                    out_hbm.at[:, pl.ds(lt * 8, 8), pl.ds(b0, CB)], sem)

            compute(0, out0)
            start_out(0, out0, so0)
            compute(1, out1)
            start_out(1, out1, so1)

            def pair(k, _):
                lt0 = 2 * k
                wait_out(out0, so0)
                compute(lt0, out0)
                start_out(lt0, out0, so0)
                wait_out(out1, so1)
                compute(lt0 + 1, out1)
                start_out(lt0 + 1, out1, so1)
                return 0

            lax.fori_loop(1, LT // 2, pair, 0)
            wait_out(out0, so0)
            compute(LT - 1, out0)
            start_out(LT - 1, out0, so0)
            wait_out(out0, so0)
            wait_out(out1, so1)

    return gather_k


_gather = _make_gather_kernel()


def kernel(input_ids, table, gamma, beta):
    nt = _normalize_table(table, gamma, beta).reshape(-1)
    ids_flat = input_ids.reshape(-1).astype(jnp.int32)
    out_t = _gather(nt, ids_flat)
    return jnp.transpose(out_t, (2, 1, 0))[:NB]


# speedup vs baseline: 50.8605x; 1.1781x over previous
import functools

import jax
import jax.numpy as jnp
from jax import lax
from jax.experimental import pallas as pl
from jax.experimental.pallas import tpu as pltpu
from jax.experimental.pallas import tpu_sc as plsc

B, Lseq, V, D = 16384, 200, 100, 10
DP = 16
NB = B - 2
NW = 32
LANES = 16

CB = 128
NBLK = B // CB
BPT = NBLK // NW
LT = Lseq // 8
SBS = CB // LANES


def _normalize_table(table, gamma, beta):
    tpad = jnp.zeros((V, DP), jnp.float32).at[:, :D].set(table)
    gpad = jnp.zeros((1, DP), jnp.float32).at[0, :D].set(gamma)
    bpad = jnp.zeros((1, DP), jnp.float32).at[0, :D].set(beta)

    def body(t_ref, g_ref, b_ref, o_ref):
        x = t_ref[...]
        mean = jnp.sum(x, axis=-1, keepdims=True) * (1.0 / D)
        mask = lax.broadcasted_iota(jnp.int32, (V, DP), 1) < D
        cen = jnp.where(mask, x - mean, 0.0)
        var = jnp.sum(cen * cen, axis=-1, keepdims=True) * (1.0 / D)
        r = lax.rsqrt(var + 1e-5)
        o_ref[...] = cen * r * g_ref[...] + b_ref[...]

    return pl.pallas_call(
        body,
        out_shape=jax.ShapeDtypeStruct((V, DP), jnp.float32),
    )(tpad, gpad, bpad)


def _make_gather_kernel():
    mesh = plsc.VectorSubcoreMesh(core_axis_name="c", subcore_axis_name="s")

    @functools.partial(
        pl.kernel,
        out_type=jax.ShapeDtypeStruct((D, Lseq, B), jnp.float32),
        mesh=mesh,
        compiler_params=pltpu.CompilerParams(needs_layout_passes=False),
        scratch_types=[
            pltpu.VMEM((V * DP,), jnp.float32),
            pltpu.VMEM((CB + 8, Lseq), jnp.int32),
            pltpu.VMEM((D, 8, CB), jnp.float32),
            pltpu.VMEM((D, 8, CB), jnp.float32),
            pltpu.SemaphoreType.DMA,
            pltpu.SemaphoreType.DMA,
        ],
    )
    def gather_k(nt_hbm, ids_hbm, out_hbm, nt_v, ids_v, out0, out1, so0, so1):
        wid = lax.axis_index("s") * 2 + lax.axis_index("c")
        pltpu.sync_copy(nt_hbm, nt_v)
        iota = lax.iota(jnp.int32, LANES)

        def wait_out(out_v, sem):
            pltpu.make_async_copy(
                out_v, out_hbm.at[:, pl.ds(0, 8), pl.ds(0, CB)], sem).wait()

        for bb in range(BPT):
            blk = wid * BPT + bb
            b0 = blk * CB
            start_row = pl.multiple_of(jnp.minimum(b0, B - (CB + 8)), 8)
            rshift = b0 + 1 - start_row
            pltpu.sync_copy(ids_hbm.at[pl.ds(start_row, CB + 8), :], ids_v)

            def compute(lt, out_v):
                @functools.partial(plsc.parallel_loop, 0, 8, unroll=2)
                def l8_body(l8):
                    l = lt * 8 + l8
                    lv = jnp.full((LANES,), 0, jnp.int32) + l
                    for sb in range(SBS):
                        bv = jnp.minimum(iota + (sb * LANES + rshift), CB + 7)
                        idsg = plsc.load_gather(ids_v, [bv, lv])
                        rowb = idsg * DP
                        for f in range(D):
                            vals = plsc.load_gather(nt_v, [rowb + f])
                            out_v[f, l8, pl.ds(sb * LANES, LANES)] = vals

            def start_out(lt, out_v, sem):
                pltpu.async_copy(
                    out_v,
                    out_hbm.at[:, pl.ds(lt * 8, 8), pl.ds(b0, CB)], sem)

            compute(0, out0)
            start_out(0, out0, so0)
            compute(1, out1)
            start_out(1, out1, so1)

            def pair(k, _):
                lt0 = 2 * k
                wait_out(out0, so0)
                compute(lt0, out0)
                start_out(lt0, out0, so0)
                wait_out(out1, so1)
                compute(lt0 + 1, out1)
                start_out(lt0 + 1, out1, so1)
                return 0

            lax.fori_loop(1, LT // 2, pair, 0)
            wait_out(out0, so0)
            compute(LT - 1, out0)
            start_out(LT - 1, out0, so0)
            wait_out(out0, so0)
            wait_out(out1, so1)

    return gather_k


_gather = _make_gather_kernel()


def kernel(input_ids, table, gamma, beta):
    nt = _normalize_table(table, gamma, beta).reshape(-1)
    out_t = _gather(nt, input_ids.astype(jnp.int32))
    return jnp.transpose(out_t, (2, 1, 0))[:NB]
